# baseline (device time: 37954 ns/iter reference)
import jax
import jax.numpy as jnp
from jax import lax
from jax.experimental import pallas as pl
from jax.experimental.pallas import tpu as pltpu

N_DEV = 32
N_STAGES = 5
E_PER = 2
CAP = 3


def kernel(x, router_W, route_idx, expert_W):
    n, d = x.shape
    h = expert_W.shape[-1]

    def body(x_ref, route_ref, w_ref, out_ref,
             acc_ref, comm_ref, send_sems, recv_sems):
        me = lax.axis_index("i")

        barrier_sem = pltpu.get_barrier_semaphore()
        for k in range(N_STAGES):
            partner = me ^ (1 << k)
            pl.semaphore_signal(
                barrier_sem, inc=1,
                device_id=(partner,), device_id_type=pl.DeviceIdType.MESH,
            )

        route = route_ref[:, :]
        my_experts = E_PER * me + jax.lax.broadcasted_iota(
            jnp.int32, (1, E_PER), 1
        )
        m = (route == my_experts)
        ii = lax.broadcasted_iota(jnp.int32, (n, n), 0)
        jj = lax.broadcasted_iota(jnp.int32, (n, n), 1)
        tril = (ii >= jj).astype(jnp.float32)
        cnt = jnp.dot(tril, m.astype(jnp.float32),
                      preferred_element_type=jnp.float32)
        keep = jnp.where(cnt <= CAP, m.astype(jnp.float32), 0.0)

        partial = jnp.zeros((n, h), jnp.float32)
        for s in range(E_PER):
            xm = x_ref[:, :] * keep[:, s][:, None]
            partial += jnp.dot(xm, w_ref[s],
                               preferred_element_type=jnp.float32)
        acc_ref[:, :] = partial

        pl.semaphore_wait(barrier_sem, N_STAGES)

        for k in range(N_STAGES):
            partner = me ^ (1 << k)
            rdma = pltpu.make_async_remote_copy(
                src_ref=acc_ref,
                dst_ref=comm_ref.at[k],
                send_sem=send_sems.at[k],
                recv_sem=recv_sems.at[k],
                device_id=(partner,),
                device_id_type=pl.DeviceIdType.MESH,
            )
            rdma.start()
            rdma.wait()
            acc_ref[:, :] = acc_ref[:, :] + comm_ref[k]

        out_ref[:, :] = acc_ref[:, :]

    return pl.pallas_call(
        body,
        out_shape=jax.ShapeDtypeStruct((n, h), jnp.float32),
        in_specs=[
            pl.BlockSpec(memory_space=pltpu.VMEM),
            pl.BlockSpec(memory_space=pltpu.VMEM),
            pl.BlockSpec(memory_space=pltpu.VMEM),
        ],
        out_specs=pl.BlockSpec(memory_space=pltpu.VMEM),
        scratch_shapes=[
            pltpu.VMEM((n, h), jnp.float32),
            pltpu.VMEM((N_STAGES, n, h), jnp.float32),
            pltpu.SemaphoreType.DMA((N_STAGES,)),
            pltpu.SemaphoreType.DMA((N_STAGES,)),
        ],
        compiler_params=pltpu.CompilerParams(collective_id=0),
    )(x, route_idx, expert_W)


# device time: 24649 ns/iter; 1.5398x vs baseline; 1.5398x over previous
import jax
import jax.numpy as jnp
from jax import lax
from jax.experimental import pallas as pl
from jax.experimental.pallas import tpu as pltpu

N_DEV = 32
N_STAGES = 5
E_PER = 2
CAP = 3
SLOTS = 8
PAYW = 384


def kernel(x, router_W, route_idx, expert_W):
    n, d = x.shape
    h = expert_W.shape[-1]

    def body(x_ref, route_ref, w_ref, out_ref, comm_ref, send_sems, recv_sems):
        me = lax.axis_index("i")

        barrier_sem = pltpu.get_barrier_semaphore()
        for k in range(N_STAGES):
            partner = me ^ (1 << k)
            pl.semaphore_signal(
                barrier_sem, inc=1,
                device_id=(partner,), device_id_type=pl.DeviceIdType.MESH,
            )

        route = route_ref[:, :]
        my_experts = E_PER * me + lax.broadcasted_iota(
            jnp.int32, (1, E_PER), 1
        )
        m = (route == my_experts).astype(jnp.float32)
        ii = lax.broadcasted_iota(jnp.int32, (n, n), 0)
        jj = lax.broadcasted_iota(jnp.int32, (n, n), 1)
        tril = (ii >= jj).astype(jnp.float32)
        cnt = jnp.dot(tril, m, preferred_element_type=jnp.float32)

        per = SLOTS // E_PER
        cntrep = jnp.concatenate(
            [cnt[:, s:s + 1] for s in range(E_PER) for _ in range(per)], axis=1
        )
        mrep = jnp.concatenate(
            [m[:, s:s + 1] for s in range(E_PER) for _ in range(per)], axis=1
        )
        r = lax.broadcasted_iota(jnp.int32, (1, SLOTS), 1) % per
        rtarget = jnp.where(r >= CAP, 0, r + 1).astype(jnp.float32)
        s_t = jnp.where(cntrep == rtarget, mrep, 0.0)

        contract0 = (((0,), (0,)), ((), ()))
        xc = lax.dot_general(s_t, x_ref[:, :], contract0,
                             preferred_element_type=jnp.float32)
        iota_col = lax.broadcasted_iota(jnp.int32, (n, 1), 0).astype(jnp.float32)
        tok = lax.dot_general(s_t, iota_col, contract0,
                              preferred_element_type=jnp.float32)

        out_rows = jnp.concatenate(
            [xc[s * per:(s + 1) * per, :] @ w_ref[s] for s in range(E_PER)],
            axis=0,
        )
        payload = jnp.concatenate(
            [out_rows, tok, jnp.zeros((SLOTS, PAYW - h - 1), jnp.float32)],
            axis=1,
        )
        comm_ref[pl.ds(me * SLOTS, SLOTS), :] = payload

        pl.semaphore_wait(barrier_sem, N_STAGES)

        for k in range(N_STAGES):
            g = 1 << k
            partner = me ^ g
            cur_base = (me // g) * g
            rows = SLOTS * g
            my_block = pl.ds(cur_base * SLOTS, rows)
            partner_block = pl.ds((cur_base ^ g) * SLOTS, rows)
            send = pltpu.make_async_remote_copy(
                src_ref=comm_ref.at[my_block],
                dst_ref=comm_ref.at[my_block],
                send_sem=send_sems.at[k],
                recv_sem=recv_sems.at[k],
                device_id=(partner,),
                device_id_type=pl.DeviceIdType.MESH,
            )
            recv = pltpu.make_async_remote_copy(
                src_ref=comm_ref.at[partner_block],
                dst_ref=comm_ref.at[partner_block],
                send_sem=send_sems.at[k],
                recv_sem=recv_sems.at[k],
                device_id=(partner,),
                device_id_type=pl.DeviceIdType.MESH,
            )
            send.start()
            recv.wait_recv()
            send.wait_send()

        gath = comm_ref[:, 0:h]
        tok_all = comm_ref[:, h:h + 1]
        tcols = lax.broadcasted_iota(
            jnp.int32, (N_DEV * SLOTS, n), 1
        ).astype(jnp.float32)
        onehot = (tok_all == tcols).astype(jnp.float32)
        out_ref[:, :] = lax.dot_general(onehot, gath, contract0,
                                        preferred_element_type=jnp.float32)

    return pl.pallas_call(
        body,
        out_shape=jax.ShapeDtypeStruct((n, h), jnp.float32),
        in_specs=[
            pl.BlockSpec(memory_space=pltpu.VMEM),
            pl.BlockSpec(memory_space=pltpu.VMEM),
            pl.BlockSpec(memory_space=pltpu.VMEM),
        ],
        out_specs=pl.BlockSpec(memory_space=pltpu.VMEM),
        scratch_shapes=[
            pltpu.VMEM((N_DEV * SLOTS, PAYW), jnp.float32),
            pltpu.SemaphoreType.DMA((N_STAGES,)),
            pltpu.SemaphoreType.DMA((N_STAGES,)),
        ],
        compiler_params=pltpu.CompilerParams(collective_id=0),
    )(x, route_idx, expert_W)


# device time: 17245 ns/iter; 2.2009x vs baseline; 1.4293x over previous
import jax
import jax.numpy as jnp
from jax import lax
from jax.experimental import pallas as pl
from jax.experimental.pallas import tpu as pltpu

N_DEV = 32
N_STAGES = 5
E_PER = 2
CAP = 3
SLOTS = 8
PAYW = 384


def kernel(x, router_W, route_idx, expert_W):
    n, d = x.shape
    h = expert_W.shape[-1]

    def body(x_ref, route_ref, w_ref, out_ref, comm_ref, send_sems, recv_sems):
        me = lax.axis_index("i")

        barrier_sem = pltpu.get_barrier_semaphore()
        for o in range(1, N_DEV):
            peer = lax.rem(me + o, N_DEV)
            pl.semaphore_signal(
                barrier_sem, inc=1,
                device_id=(peer,), device_id_type=pl.DeviceIdType.MESH,
            )

        route = route_ref[:, :]
        my_experts = E_PER * me + lax.broadcasted_iota(
            jnp.int32, (1, E_PER), 1
        )
        m = (route == my_experts).astype(jnp.float32)
        ii = lax.broadcasted_iota(jnp.int32, (n, n), 0)
        jj = lax.broadcasted_iota(jnp.int32, (n, n), 1)
        tril = (ii >= jj).astype(jnp.float32)
        cnt = jnp.dot(tril, m, preferred_element_type=jnp.float32)

        per = SLOTS // E_PER
        cntrep = jnp.concatenate(
            [cnt[:, s:s + 1] for s in range(E_PER) for _ in range(per)], axis=1
        )
        mrep = jnp.concatenate(
            [m[:, s:s + 1] for s in range(E_PER) for _ in range(per)], axis=1
        )
        r = lax.broadcasted_iota(jnp.int32, (1, SLOTS), 1) % per
        rtarget = jnp.where(r >= CAP, 0, r + 1).astype(jnp.float32)
        s_t = jnp.where(cntrep == rtarget, mrep, 0.0)

        contract0 = (((0,), (0,)), ((), ()))
        xc = lax.dot_general(s_t, x_ref[:, :], contract0,
                             preferred_element_type=jnp.float32)
        iota_col = lax.broadcasted_iota(jnp.int32, (n, 1), 0).astype(jnp.float32)
        tok = lax.dot_general(s_t, iota_col, contract0,
                              preferred_element_type=jnp.float32)

        out_rows = jnp.concatenate(
            [xc[s * per:(s + 1) * per, :] @ w_ref[s] for s in range(E_PER)],
            axis=0,
        )
        payload = jnp.concatenate(
            [out_rows, tok, jnp.zeros((SLOTS, PAYW - h - 1), jnp.float32)],
            axis=1,
        )
        comm_ref[pl.ds(me * SLOTS, SLOTS), :] = payload

        pl.semaphore_wait(barrier_sem, N_DEV - 1)

        my_block = pl.ds(me * SLOTS, SLOTS)
        sends = []
        for o in range(1, N_DEV):
            peer = lax.rem(me + o, N_DEV)
            send = pltpu.make_async_remote_copy(
                src_ref=comm_ref.at[my_block],
                dst_ref=comm_ref.at[my_block],
                send_sem=send_sems.at[o - 1],
                recv_sem=recv_sems.at[o - 1],
                device_id=(peer,),
                device_id_type=pl.DeviceIdType.MESH,
            )
            send.start()
            sends.append(send)
        for o in range(1, N_DEV):
            sender = lax.rem(me - o + N_DEV, N_DEV)
            sender_block = pl.ds(sender * SLOTS, SLOTS)
            recv = pltpu.make_async_remote_copy(
                src_ref=comm_ref.at[sender_block],
                dst_ref=comm_ref.at[sender_block],
                send_sem=send_sems.at[o - 1],
                recv_sem=recv_sems.at[o - 1],
                device_id=(sender,),
                device_id_type=pl.DeviceIdType.MESH,
            )
            recv.wait_recv()
        for send in sends:
            send.wait_send()

        gath = comm_ref[:, 0:h]
        tok_all = comm_ref[:, h:h + 1]
        tcols = lax.broadcasted_iota(
            jnp.int32, (N_DEV * SLOTS, n), 1
        ).astype(jnp.float32)
        onehot = (tok_all == tcols).astype(jnp.float32)
        out_ref[:, :] = lax.dot_general(onehot, gath, contract0,
                                        preferred_element_type=jnp.float32)

    return pl.pallas_call(
        body,
        out_shape=jax.ShapeDtypeStruct((n, h), jnp.float32),
        in_specs=[
            pl.BlockSpec(memory_space=pltpu.VMEM),
            pl.BlockSpec(memory_space=pltpu.VMEM),
            pl.BlockSpec(memory_space=pltpu.VMEM),
        ],
        out_specs=pl.BlockSpec(memory_space=pltpu.VMEM),
        scratch_shapes=[
            pltpu.VMEM((N_DEV * SLOTS, PAYW), jnp.float32),
            pltpu.SemaphoreType.DMA((N_DEV - 1,)),
            pltpu.SemaphoreType.DMA((N_DEV - 1,)),
        ],
        compiler_params=pltpu.CompilerParams(collective_id=0),
    )(x, route_idx, expert_W)


# device time: 15897 ns/iter; 2.3875x vs baseline; 1.0848x over previous
import jax
import jax.numpy as jnp
from jax import lax
from jax.experimental import pallas as pl
from jax.experimental.pallas import tpu as pltpu

N_DEV = 32
N_EXP = 64
E_PER = 2
CAP = 3
SLOTS = 8


def kernel(x, router_W, route_idx, expert_W):
    n, d = x.shape
    h = expert_W.shape[-1]
    rows_total = N_DEV * SLOTS

    def body(x_ref, route_ref, w_ref, out_ref, comm_ref, send_sems, recv_sems):
        me = lax.axis_index("i")

        barrier_sem = pltpu.get_barrier_semaphore()
        for o in range(1, N_DEV):
            peer = lax.rem(me + o, N_DEV)
            pl.semaphore_signal(
                barrier_sem, inc=1,
                device_id=(peer,), device_id_type=pl.DeviceIdType.MESH,
            )

        route = route_ref[:, :]
        my_experts = E_PER * me + lax.broadcasted_iota(
            jnp.int32, (1, E_PER), 1
        )
        m = (route == my_experts).astype(jnp.float32)
        ii = lax.broadcasted_iota(jnp.int32, (n, n), 0)
        jj = lax.broadcasted_iota(jnp.int32, (n, n), 1)
        tril = (ii >= jj).astype(jnp.float32)
        cnt = jnp.dot(tril, m, preferred_element_type=jnp.float32)

        per = SLOTS // E_PER
        cntrep = jnp.concatenate(
            [cnt[:, s:s + 1] for s in range(E_PER) for _ in range(per)], axis=1
        )
        mrep = jnp.concatenate(
            [m[:, s:s + 1] for s in range(E_PER) for _ in range(per)], axis=1
        )
        r = lax.broadcasted_iota(jnp.int32, (1, SLOTS), 1) % per
        rtarget = jnp.where(r >= CAP, 0, r + 1).astype(jnp.float32)
        s_t = jnp.where(cntrep == rtarget, mrep, 0.0)

        contract0 = (((0,), (0,)), ((), ()))
        xc = lax.dot_general(s_t, x_ref[:, :], contract0,
                             preferred_element_type=jnp.float32)
        payload = jnp.concatenate(
            [xc[s * per:(s + 1) * per, :] @ w_ref[s] for s in range(E_PER)],
            axis=0,
        )
        comm_ref[pl.ds(me * SLOTS, SLOTS), :] = payload

        pl.semaphore_wait(barrier_sem, N_DEV - 1)

        my_block = pl.ds(me * SLOTS, SLOTS)
        sends = []
        for o in range(1, N_DEV):
            peer = lax.rem(me + o, N_DEV)
            send = pltpu.make_async_remote_copy(
                src_ref=comm_ref.at[my_block],
                dst_ref=comm_ref.at[my_block],
                send_sem=send_sems.at[o - 1],
                recv_sem=recv_sems.at[o - 1],
                device_id=(peer,),
                device_id_type=pl.DeviceIdType.MESH,
            )
            send.start()
            sends.append(send)

        e_cols = lax.broadcasted_iota(jnp.int32, (n, N_EXP), 1)
        m_all = (route == e_cols).astype(jnp.float32)
        cnt_all = jnp.dot(tril, m_all,
                          preferred_element_type=jnp.float32)
        keep_all = jnp.where(cnt_all <= CAP, m_all, 0.0)
        grow = keep_all * (4.0 * e_cols.astype(jnp.float32) + cnt_all - 1.0)
        gidx = jnp.sum(grow, axis=1, keepdims=True)
        valid = jnp.sum(keep_all, axis=1, keepdims=True)
        gidx = jnp.where(valid > 0.0, gidx, -1.0)
        grows = lax.broadcasted_iota(
            jnp.int32, (n, rows_total), 1
        ).astype(jnp.float32)
        scatter = (gidx == grows).astype(jnp.float32)

        for o in range(1, N_DEV):
            sender = lax.rem(me - o + N_DEV, N_DEV)
            sender_block = pl.ds(sender * SLOTS, SLOTS)
            recv = pltpu.make_async_remote_copy(
                src_ref=comm_ref.at[sender_block],
                dst_ref=comm_ref.at[sender_block],
                send_sem=send_sems.at[o - 1],
                recv_sem=recv_sems.at[o - 1],
                device_id=(sender,),
                device_id_type=pl.DeviceIdType.MESH,
            )
            recv.wait_recv()
        for send in sends:
            send.wait_send()

        out_ref[:, :] = jnp.dot(scatter, comm_ref[:, :],
                                preferred_element_type=jnp.float32)

    return pl.pallas_call(
        body,
        out_shape=jax.ShapeDtypeStruct((n, h), jnp.float32),
        in_specs=[
            pl.BlockSpec(memory_space=pltpu.VMEM),
            pl.BlockSpec(memory_space=pltpu.VMEM),
            pl.BlockSpec(memory_space=pltpu.VMEM),
        ],
        out_specs=pl.BlockSpec(memory_space=pltpu.VMEM),
        scratch_shapes=[
            pltpu.VMEM((N_DEV * SLOTS, 256), jnp.float32),
            pltpu.SemaphoreType.DMA((N_DEV - 1,)),
            pltpu.SemaphoreType.DMA((N_DEV - 1,)),
        ],
        compiler_params=pltpu.CompilerParams(collective_id=0),
    )(x, route_idx, expert_W)
